# baseline (speedup 1.0000x reference)
import functools

import jax
import jax.numpy as jnp
from jax import lax
from jax.experimental import pallas as pl
from jax.experimental.pallas import tpu as pltpu
from jax.experimental.pallas import tpu_sc as plsc

NC = 2
NS = 16
K = 128


_SP = 40


@functools.lru_cache(maxsize=None)
def _make_sc_scatter(npad: int, fc: int, n0: int, n1: int, gather: bool = True,
                     dtype=jnp.float32, fast_core: int = 0):
    rows_per_tile = npad // NS
    assert npad % NS == 0 and rows_per_tile % 8 == 0
    assert n0 % _SP == 0 and (n1 % _SP == 0 or n1 == 0)
    ncores_out = 1 if n1 == 0 else NC
    mesh = plsc.VectorSubcoreMesh(core_axis_name="c", subcore_axis_name="s")

    @functools.partial(
        pl.kernel,
        mesh=mesh,
        out_type=jax.ShapeDtypeStruct((ncores_out, npad, fc), dtype),
        scratch_types=[
            pltpu.VMEM((_SP, K), jnp.int32),
            pltpu.VMEM((_SP, K), jnp.int32),
            pltpu.VMEM((K, fc), dtype),
            pltpu.VMEM((K, fc), dtype),
            pltpu.VMEM_SHARED((npad, fc), dtype),
            pltpu.SemaphoreType.DMA,
            pltpu.SemaphoreType.DMA,
        ],
    )
    def sc_scatter(table_hbm, src_hbm, dst_hbm, zeros_hbm, out_hbm,
                   src_v, dst_v, buf0, buf1, acc_sh, sem0, sem1):
        c = lax.axis_index("c")
        s = lax.axis_index("s")
        r0 = s * rows_per_tile
        bufs = (buf0, buf1)
        sems = (sem0, sem1)

        def run_phase(row_base, do_gather):
            if do_gather:
                pltpu.sync_copy(src_hbm.at[pl.ds(row_base, _SP)], src_v)
            pltpu.sync_copy(dst_hbm.at[pl.ds(row_base, _SP)], dst_v)
            if do_gather:
                pltpu.async_copy(table_hbm.at[src_v.at[0]], buf0, sem0)

                def body(r, carry):
                    for b in range(2):
                        j = 2 * r + b

                        @pl.when(j + 1 < _SP)
                        def _():
                            pltpu.async_copy(table_hbm.at[src_v.at[j + 1]],
                                             bufs[1 - b], sems[1 - b])

                        pltpu.make_async_copy(table_hbm.at[src_v.at[j]],
                                              bufs[b], sems[b]).wait()
                        pltpu.sync_copy(bufs[b], acc_sh.at[dst_v.at[j]], add=True)
                    return carry

                lax.fori_loop(0, _SP // 2, body, 0)
            else:
                def body(j, carry):
                    pltpu.sync_copy(buf0, acc_sh.at[dst_v.at[j]], add=True)
                    return carry

                lax.fori_loop(0, _SP, body, 0)

        def zero_acc():
            pltpu.sync_copy(zeros_hbm.at[pl.ds(r0, rows_per_tile)],
                            acc_sh.at[pl.ds(r0, rows_per_tile)])

        if n1 == 0:
            @pl.when(c == fast_core)
            def _():
                zero_acc()
                plsc.subcore_barrier()
                row0 = s * n0
                for p in range(n0 // _SP):
                    run_phase(row0 + p * _SP, gather)
                plsc.subcore_barrier()
                pltpu.sync_copy(acc_sh.at[pl.ds(r0, rows_per_tile)],
                                out_hbm.at[0, pl.ds(r0, rows_per_tile)])
        else:
            zero_acc()
            if not gather:
                pltpu.sync_copy(table_hbm.at[pl.ds(0, K)], buf0)
            plsc.subcore_barrier()
            row0 = lax.select(c == 0, s * n0, NS * n0 + s * n1)
            nph = n0 // _SP
            for p in range(nph):
                run_phase(row0 + p * _SP, gather)
            plsc.subcore_barrier()
            pltpu.sync_copy(acc_sh.at[pl.ds(r0, rows_per_tile)],
                            out_hbm.at[c, pl.ds(r0, rows_per_tile)])

    return sc_scatter


_R = 632


def _row_spec(*block):
    return pl.BlockSpec(block, lambda i: (0,) * (len(block) - 2) + (i, 0))


def _full_spec(*block):
    return pl.BlockSpec(block, lambda i: (0,) * len(block))


def _tc_call(body, npad, in_specs, out_shapes, out_specs):
    return pl.pallas_call(
        body,
        grid=(npad // _R,),
        in_specs=in_specs,
        out_specs=out_specs,
        out_shape=out_shapes,
    )


def _dinv(da, db):
    return lax.rsqrt(da + db + 1.0)


def _t1_body(x_ref, w_ref, da_ref, db_ref, outa_ref, outb_ref):
    dinv = _dinv(da_ref[...], db_ref[...])
    g = dinv * jnp.dot(x_ref[...], w_ref[...], preferred_element_type=jnp.float32)
    outa_ref[...] = g[:, :128]
    outb_ref[...] = g[:, 128:]


def _t2_body(sa_ref, sb_ref, ga_ref, gb_ref, da_ref, db_ref, b_ref, w_ref, out_ref):
    dinv = _dinv(da_ref[...], db_ref[...])
    b = b_ref[...]
    ha = jnp.maximum(dinv * (sa_ref[0] + ga_ref[...]) + b[:, :128], 0.0)
    hb = jnp.maximum(dinv * (sb_ref[0] + gb_ref[...]) + b[:, 128:], 0.0)
    h = jnp.concatenate([ha, hb], axis=1)
    out_ref[...] = dinv * jnp.dot(h, w_ref[...], preferred_element_type=jnp.float32)


def _t3_body(s_ref, g_ref, da_ref, db_ref, b_ref, w_ref, out_ref):
    dinv = _dinv(da_ref[...], db_ref[...])
    h = jnp.maximum(dinv * (s_ref[0] + g_ref[...]) + b_ref[...], 0.0)
    out_ref[...] = dinv * jnp.dot(h, w_ref[...], preferred_element_type=jnp.float32)


def _t4_body(s_ref, g_ref, da_ref, db_ref, b_ref, out_ref):
    dinv = _dinv(da_ref[...], db_ref[...])
    t = (s_ref[0] + g_ref[...])[:, :64]
    out_ref[...] = dinv * t + b_ref[...]


def kernel(x, edge_index, W1, b1, W2, b2, W3, b3):
    N, F0 = x.shape
    E = edge_index.shape[1]
    npad = 10112
    assert npad >= N + 1 and npad % _R == 0 and npad % (NS * 8) == 0
    per_pair = -(-E // (NS * K))
    per_pair = ((per_pair + 2 * _SP - 1) // (2 * _SP)) * (2 * _SP)
    neven = per_pair // 2
    nrows = NS * per_pair
    epad = nrows * K

    xp = jnp.zeros((npad, F0), jnp.float32).at[:N].set(x)
    src = jnp.full((epad,), N, jnp.int32).at[:E].set(edge_index[0].astype(jnp.int32))
    dst = jnp.full((epad,), N, jnp.int32).at[:E].set(edge_index[1].astype(jnp.int32))
    src2d = src.reshape(nrows, K)
    dst2d = dst.reshape(nrows, K)

    def sc_pass(table, fc, gather=True):
        z = jnp.zeros((npad, fc), table.dtype)
        if gather:
            n0, n1 = per_pair, 0
        else:
            n0, n1 = neven, neven
        return _make_sc_scatter(npad, fc, n0, n1, gather, table.dtype,
                                fast_core=1)(table, src2d, dst2d, z)

    deg = sc_pass(jnp.ones((K, 128), jnp.float32), 128, gather=False)
    da = deg[0, :, :1]
    db = deg[1, :, :1]

    dspec = _row_spec(_R, 1)

    g1a, g1b = _tc_call(
        _t1_body, npad,
        [_row_spec(_R, 256), _full_spec(256, 256), dspec, dspec],
        [jax.ShapeDtypeStruct((npad, 128), jnp.float32)] * 2,
        [_row_spec(_R, 128)] * 2,
    )(xp, W1, da, db)
    s1a = sc_pass(g1a, 128)
    s1b = sc_pass(g1b, 128)

    g2 = _tc_call(
        _t2_body, npad,
        [_row_spec(1, _R, 128)] * 2 + [_row_spec(_R, 128)] * 2 + [dspec, dspec,
         _full_spec(1, 256), _full_spec(256, 128)],
        jax.ShapeDtypeStruct((npad, 128), jnp.float32),
        _row_spec(_R, 128),
    )(s1a, s1b, g1a, g1b, da, db, b1.reshape(1, 256), W2)
    s2 = sc_pass(g2, 128)

    W3p = jnp.zeros((128, 128), jnp.float32).at[:, :64].set(W3)
    g3 = _tc_call(
        _t3_body, npad,
        [_row_spec(1, _R, 128), _row_spec(_R, 128), dspec, dspec,
         _full_spec(1, 128), _full_spec(128, 128)],
        jax.ShapeDtypeStruct((npad, 128), jnp.float32),
        _row_spec(_R, 128),
    )(s2, g2, da, db, b2.reshape(1, 128), W3p)
    s3 = sc_pass(g3, 128)

    out = _tc_call(
        _t4_body, npad,
        [_row_spec(1, _R, 128), _row_spec(_R, 128), dspec, dspec, _full_spec(1, 64)],
        jax.ShapeDtypeStruct((npad, 64), jnp.float32),
        _row_spec(_R, 64),
    )(s3, g3, da, db, b3.reshape(1, 64))
    return out[:N]


# speedup vs baseline: 1.2326x; 1.2326x over previous
import functools

import jax
import jax.numpy as jnp
from jax import lax
from jax.experimental import pallas as pl
from jax.experimental.pallas import tpu as pltpu
from jax.experimental.pallas import tpu_sc as plsc

NC = 2
NS = 16
K = 128


@functools.lru_cache(maxsize=None)
def _make_sc_scatter(npad: int, fc: int, n0: int, n1: int, gather: bool = True,
                     dtype=jnp.float32):
    rows_per_tile = npad // NS
    assert npad % NS == 0 and rows_per_tile % 8 == 0
    assert n0 % 8 == 0 and n1 % 8 == 0
    nmax = max(n0, n1)
    mesh = plsc.VectorSubcoreMesh(core_axis_name="c", subcore_axis_name="s")

    @functools.partial(
        pl.kernel,
        mesh=mesh,
        out_type=jax.ShapeDtypeStruct((NC, npad, fc), dtype),
        scratch_types=[
            pltpu.VMEM((nmax, K), jnp.int32),
            pltpu.VMEM((nmax, K), jnp.int32),
            pltpu.VMEM((K, fc), dtype),
            pltpu.VMEM((K, fc), dtype),
            pltpu.VMEM_SHARED((npad, fc), dtype),
            pltpu.SemaphoreType.DMA,
            pltpu.SemaphoreType.DMA,
        ],
    )
    def sc_scatter(table_hbm, src_hbm, dst_hbm, zeros_hbm, out_hbm,
                   src_v, dst_v, buf0, buf1, acc_sh, sem0, sem1):
        c = lax.axis_index("c")
        s = lax.axis_index("s")
        nc = lax.select(c == 0, n0, n1)

        r0 = s * rows_per_tile
        pltpu.sync_copy(zeros_hbm.at[pl.ds(r0, rows_per_tile)],
                        acc_sh.at[pl.ds(r0, rows_per_tile)])
        row0 = lax.select(c == 0, s * n0, NS * n0 + s * n1)
        if gather:
            pltpu.sync_copy(src_hbm.at[pl.ds(row0, nmax)], src_v)
        else:
            pltpu.sync_copy(table_hbm.at[pl.ds(0, K)], buf0)
        pltpu.sync_copy(dst_hbm.at[pl.ds(row0, nmax)], dst_v)
        plsc.subcore_barrier()

        if gather:
            bufs = (buf0, buf1)
            sems = (sem0, sem1)

            pltpu.async_copy(table_hbm.at[src_v.at[0]], buf0, sem0)

            def body(r, carry):
                for b in range(2):
                    j = 2 * r + b

                    @pl.when(j + 1 < nc)
                    def _():
                        pltpu.async_copy(
                            table_hbm.at[src_v.at[j + 1]], bufs[1 - b], sems[1 - b])

                    pltpu.make_async_copy(
                        table_hbm.at[src_v.at[j]], bufs[b], sems[b]).wait()
                    pltpu.sync_copy(bufs[b], acc_sh.at[dst_v.at[j]], add=True)
                return carry

            lax.fori_loop(0, nc // 2, body, 0)
        else:
            def body(j, carry):
                pltpu.sync_copy(buf0, acc_sh.at[dst_v.at[j]], add=True)
                return carry

            lax.fori_loop(0, nc, body, 0)
        plsc.subcore_barrier()

        pltpu.sync_copy(acc_sh.at[pl.ds(r0, rows_per_tile)],
                        out_hbm.at[c, pl.ds(r0, rows_per_tile)])

    return sc_scatter


_R = 632


def _row_spec(*block):
    return pl.BlockSpec(block, lambda i: (0,) * (len(block) - 2) + (i, 0))


def _full_spec(*block):
    return pl.BlockSpec(block, lambda i: (0,) * len(block))


def _tc_call(body, npad, in_specs, out_shapes, out_specs):
    return pl.pallas_call(
        body,
        grid=(npad // _R,),
        in_specs=in_specs,
        out_specs=out_specs,
        out_shape=out_shapes,
    )


def _dinv(da, db):
    return lax.rsqrt(da + db + 1.0)


def _t1_body(x_ref, w_ref, da_ref, db_ref, outa_ref, outb_ref):
    dinv = _dinv(da_ref[...], db_ref[...])
    g = dinv * jnp.dot(x_ref[...], w_ref[...], preferred_element_type=jnp.float32)
    outa_ref[...] = g[:, :128]
    outb_ref[...] = g[:, 128:]


def _t2_body(sa_ref, sb_ref, ga_ref, gb_ref, da_ref, db_ref, b_ref, w_ref, out_ref):
    dinv = _dinv(da_ref[...], db_ref[...])
    b = b_ref[...]
    ha = jnp.maximum(dinv * (sa_ref[0] + sa_ref[1] + ga_ref[...]) + b[:, :128], 0.0)
    hb = jnp.maximum(dinv * (sb_ref[0] + sb_ref[1] + gb_ref[...]) + b[:, 128:], 0.0)
    h = jnp.concatenate([ha, hb], axis=1)
    out_ref[...] = dinv * jnp.dot(h, w_ref[...], preferred_element_type=jnp.float32)


def _t3_body(s_ref, g_ref, da_ref, db_ref, b_ref, w_ref, out_ref):
    dinv = _dinv(da_ref[...], db_ref[...])
    h = jnp.maximum(dinv * (s_ref[0] + s_ref[1] + g_ref[...]) + b_ref[...], 0.0)
    out_ref[...] = dinv * jnp.dot(h, w_ref[...], preferred_element_type=jnp.float32)


def _t4_body(s_ref, g_ref, da_ref, db_ref, b_ref, out_ref):
    dinv = _dinv(da_ref[...], db_ref[...])
    t = (s_ref[0] + s_ref[1] + g_ref[...])[:, :64]
    out_ref[...] = dinv * t + b_ref[...]


def kernel(x, edge_index, W1, b1, W2, b2, W3, b3):
    N, F0 = x.shape
    E = edge_index.shape[1]
    npad = 10112
    assert npad >= N + 1 and npad % _R == 0 and npad % (NS * 8) == 0
    per_pair = -(-E // (NS * K))
    per_pair = ((per_pair + 15) // 16) * 16
    n0g = ((per_pair * 4 // 5) + 7) // 8 * 8
    n1g = per_pair - n0g
    neven = per_pair // 2
    nrows = NS * n0g + (NS - 1) * n1g + max(n0g, n1g)
    nrows = max(nrows, NS * per_pair)
    epad = nrows * K

    xp = jnp.zeros((npad, F0), jnp.float32).at[:N].set(x)
    src = jnp.full((epad,), N, jnp.int32).at[:E].set(edge_index[0].astype(jnp.int32))
    dst = jnp.full((epad,), N, jnp.int32).at[:E].set(edge_index[1].astype(jnp.int32))
    src2d = src.reshape(nrows, K)
    dst2d = dst.reshape(nrows, K)

    def sc_pass(table, fc, gather=True):
        z = jnp.zeros((npad, fc), table.dtype)
        if gather:
            n0, n1 = n0g, n1g
        else:
            n0, n1 = neven, neven
        return _make_sc_scatter(npad, fc, n0, n1, gather, table.dtype)(
            table, src2d, dst2d, z)

    deg = sc_pass(jnp.ones((K, 128), jnp.float32), 128, gather=False)
    da = deg[0, :, :1]
    db = deg[1, :, :1]

    dspec = _row_spec(_R, 1)

    g1a, g1b = _tc_call(
        _t1_body, npad,
        [_row_spec(_R, 256), _full_spec(256, 256), dspec, dspec],
        [jax.ShapeDtypeStruct((npad, 128), jnp.float32)] * 2,
        [_row_spec(_R, 128)] * 2,
    )(xp, W1, da, db)
    s1a = sc_pass(g1a, 128)
    s1b = sc_pass(g1b, 128)

    g2 = _tc_call(
        _t2_body, npad,
        [_row_spec(2, _R, 128)] * 2 + [_row_spec(_R, 128)] * 2 + [dspec, dspec,
         _full_spec(1, 256), _full_spec(256, 128)],
        jax.ShapeDtypeStruct((npad, 128), jnp.float32),
        _row_spec(_R, 128),
    )(s1a, s1b, g1a, g1b, da, db, b1.reshape(1, 256), W2)
    s2 = sc_pass(g2, 128)

    W3p = jnp.zeros((128, 128), jnp.float32).at[:, :64].set(W3)
    g3 = _tc_call(
        _t3_body, npad,
        [_row_spec(2, _R, 128), _row_spec(_R, 128), dspec, dspec,
         _full_spec(1, 128), _full_spec(128, 128)],
        jax.ShapeDtypeStruct((npad, 128), jnp.float32),
        _row_spec(_R, 128),
    )(s2, g2, da, db, b2.reshape(1, 128), W3p)
    s3 = sc_pass(g3, 128)

    out = _tc_call(
        _t4_body, npad,
        [_row_spec(2, _R, 128), _row_spec(_R, 128), dspec, dspec, _full_spec(1, 64)],
        jax.ShapeDtypeStruct((npad, 64), jnp.float32),
        _row_spec(_R, 64),
    )(s3, g3, da, db, b3.reshape(1, 64))
    return out[:N]


# speedup vs baseline: 1.3536x; 1.0982x over previous
import functools

import jax
import jax.numpy as jnp
from jax import lax
from jax.experimental import pallas as pl
from jax.experimental.pallas import tpu as pltpu
from jax.experimental.pallas import tpu_sc as plsc

NC = 2
NS = 16
K = 128


@functools.lru_cache(maxsize=None)
def _make_sc_scatter(npad: int, fc: int, n0: int, n1: int, gather: bool = True,
                     dtype=jnp.float32):
    rows_per_tile = npad // NS
    assert npad % NS == 0 and rows_per_tile % 8 == 0
    assert n0 % 8 == 0 and n1 % 8 == 0
    nmax = max(n0, n1)
    nseg = 2 if nmax > 64 else 1
    sv = (nmax + nseg - 1) // nseg
    sv = ((sv + 7) // 8) * 8
    mesh = plsc.VectorSubcoreMesh(core_axis_name="c", subcore_axis_name="s")

    @functools.partial(
        pl.kernel,
        mesh=mesh,
        out_type=jax.ShapeDtypeStruct((NC, npad, fc), dtype),
        scratch_types=[
            pltpu.VMEM((sv, K), jnp.int32),
            pltpu.VMEM((nmax, K), jnp.int32),
            pltpu.VMEM((K, fc), dtype),
            pltpu.VMEM((K, fc), dtype),
            pltpu.VMEM_SHARED((npad, fc), dtype),
            pltpu.SemaphoreType.DMA,
            pltpu.SemaphoreType.DMA,
        ],
    )
    def sc_scatter(table_hbm, src_hbm, dst_hbm, zeros_hbm, out_hbm,
                   src_v, dst_v, buf0, buf1, acc_sh, sem0, sem1):
        c = lax.axis_index("c")
        s = lax.axis_index("s")
        nc = lax.select(c == 0, n0, n1)

        r0 = s * rows_per_tile
        pltpu.sync_copy(zeros_hbm.at[pl.ds(r0, rows_per_tile)],
                        acc_sh.at[pl.ds(r0, rows_per_tile)])
        row0 = lax.select(c == 0, s * n0, NS * n0 + s * n1)
        if not gather:
            pltpu.sync_copy(table_hbm.at[pl.ds(0, K)], buf0)
        pltpu.sync_copy(dst_hbm.at[pl.ds(row0, nmax)], dst_v)
        plsc.subcore_barrier()

        if gather:
            bufs = (buf0, buf1)
            sems = (sem0, sem1)

            for p in range(nseg):
                seg0 = p * sv
                trip = lax.max(0, lax.min(sv, nc - seg0))

                @pl.when(trip > 0)
                def _():
                    pltpu.sync_copy(src_hbm.at[pl.ds(row0 + seg0, sv)], src_v)
                    pltpu.async_copy(table_hbm.at[src_v.at[0]], buf0, sem0)

                    def body(r, carry):
                        for b in range(2):
                            jl = 2 * r + b

                            @pl.when(jl + 1 < trip)
                            def _():
                                pltpu.async_copy(table_hbm.at[src_v.at[jl + 1]],
                                                 bufs[1 - b], sems[1 - b])

                            pltpu.make_async_copy(
                                table_hbm.at[src_v.at[jl]], bufs[b], sems[b]).wait()
                            pltpu.sync_copy(bufs[b],
                                            acc_sh.at[dst_v.at[seg0 + jl]],
                                            add=True)
                        return carry

                    lax.fori_loop(0, trip // 2, body, 0)
        else:
            def body(j, carry):
                pltpu.sync_copy(buf0, acc_sh.at[dst_v.at[j]], add=True)
                return carry

            lax.fori_loop(0, nc, body, 0)
        plsc.subcore_barrier()

        pltpu.sync_copy(acc_sh.at[pl.ds(r0, rows_per_tile)],
                        out_hbm.at[c, pl.ds(r0, rows_per_tile)])

    return sc_scatter


_R = 632


def _row_spec(*block):
    return pl.BlockSpec(block, lambda i: (0,) * (len(block) - 2) + (i, 0))


def _full_spec(*block):
    return pl.BlockSpec(block, lambda i: (0,) * len(block))


def _tc_call(body, npad, in_specs, out_shapes, out_specs):
    return pl.pallas_call(
        body,
        grid=(npad // _R,),
        in_specs=in_specs,
        out_specs=out_specs,
        out_shape=out_shapes,
    )


def _dinv(da, db):
    return lax.rsqrt(da + db + 1.0)


def _t1_body(x_ref, w_ref, da_ref, db_ref, outa_ref, outb_ref):
    dinv = _dinv(da_ref[...], db_ref[...])
    g = dinv * jnp.dot(x_ref[...], w_ref[...], preferred_element_type=jnp.float32)
    outa_ref[...] = g[:, :128]
    outb_ref[...] = g[:, 128:]


def _t2_body(sa_ref, sb_ref, ga_ref, gb_ref, da_ref, db_ref, b_ref, w_ref, out_ref):
    dinv = _dinv(da_ref[...], db_ref[...])
    b = b_ref[...]
    ha = jnp.maximum(dinv * (sa_ref[0] + sa_ref[1] + ga_ref[...]) + b[:, :128], 0.0)
    hb = jnp.maximum(dinv * (sb_ref[0] + sb_ref[1] + gb_ref[...]) + b[:, 128:], 0.0)
    h = jnp.concatenate([ha, hb], axis=1)
    out_ref[...] = dinv * jnp.dot(h, w_ref[...], preferred_element_type=jnp.float32)


def _t3_body(s_ref, g_ref, da_ref, db_ref, b_ref, w_ref, out_ref):
    dinv = _dinv(da_ref[...], db_ref[...])
    h = jnp.maximum(dinv * (s_ref[0] + s_ref[1] + g_ref[...]) + b_ref[...], 0.0)
    out_ref[...] = dinv * jnp.dot(h, w_ref[...], preferred_element_type=jnp.float32)


def _t4_body(s_ref, g_ref, da_ref, db_ref, b_ref, out_ref):
    dinv = _dinv(da_ref[...], db_ref[...])
    t = (s_ref[0] + s_ref[1] + g_ref[...])[:, :64]
    out_ref[...] = dinv * t + b_ref[...]


def kernel(x, edge_index, W1, b1, W2, b2, W3, b3):
    N, F0 = x.shape
    E = edge_index.shape[1]
    npad = 10112
    assert npad >= N + 1 and npad % _R == 0 and npad % (NS * 8) == 0
    per_pair = -(-E // (NS * K))
    per_pair = ((per_pair + 15) // 16) * 16
    n0g = ((per_pair * 9 // 10) + 7) // 8 * 8
    n1g = per_pair - n0g
    neven = per_pair // 2
    nrows = NS * n0g + (NS - 1) * n1g + max(n0g, n1g)
    nrows = max(nrows, NS * per_pair)
    epad = nrows * K

    xp = jnp.zeros((npad, F0), jnp.float32).at[:N].set(x)
    src = jnp.full((epad,), N, jnp.int32).at[:E].set(edge_index[0].astype(jnp.int32))
    dst = jnp.full((epad,), N, jnp.int32).at[:E].set(edge_index[1].astype(jnp.int32))
    src2d = src.reshape(nrows, K)
    dst2d = dst.reshape(nrows, K)

    def sc_pass(table, fc, gather=True):
        z = jnp.zeros((npad, fc), table.dtype)
        if gather:
            n0, n1 = n0g, n1g
        else:
            n0, n1 = neven, neven
        return _make_sc_scatter(npad, fc, n0, n1, gather, table.dtype)(
            table, src2d, dst2d, z)

    deg = sc_pass(jnp.ones((K, 128), jnp.float32), 128, gather=False)
    da = deg[0, :, :1]
    db = deg[1, :, :1]

    dspec = _row_spec(_R, 1)

    g1a, g1b = _tc_call(
        _t1_body, npad,
        [_row_spec(_R, 256), _full_spec(256, 256), dspec, dspec],
        [jax.ShapeDtypeStruct((npad, 128), jnp.float32)] * 2,
        [_row_spec(_R, 128)] * 2,
    )(xp, W1, da, db)
    s1a = sc_pass(g1a, 128)
    s1b = sc_pass(g1b, 128)

    g2 = _tc_call(
        _t2_body, npad,
        [_row_spec(2, _R, 128)] * 2 + [_row_spec(_R, 128)] * 2 + [dspec, dspec,
         _full_spec(1, 256), _full_spec(256, 128)],
        jax.ShapeDtypeStruct((npad, 128), jnp.float32),
        _row_spec(_R, 128),
    )(s1a, s1b, g1a, g1b, da, db, b1.reshape(1, 256), W2)
    s2 = sc_pass(g2, 128)

    W3p = jnp.zeros((128, 128), jnp.float32).at[:, :64].set(W3)
    g3 = _tc_call(
        _t3_body, npad,
        [_row_spec(2, _R, 128), _row_spec(_R, 128), dspec, dspec,
         _full_spec(1, 128), _full_spec(128, 128)],
        jax.ShapeDtypeStruct((npad, 128), jnp.float32),
        _row_spec(_R, 128),
    )(s2, g2, da, db, b2.reshape(1, 128), W3p)
    s3 = sc_pass(g3, 128)

    out = _tc_call(
        _t4_body, npad,
        [_row_spec(2, _R, 128), _row_spec(_R, 128), dspec, dspec, _full_spec(1, 64)],
        jax.ShapeDtypeStruct((npad, 64), jnp.float32),
        _row_spec(_R, 64),
    )(s3, g3, da, db, b3.reshape(1, 64))
    return out[:N]
